# baseline (speedup 1.0000x reference)
import functools

import jax
import jax.numpy as jnp
from jax import lax
from jax.experimental import pallas as pl
from jax.experimental.pallas import tpu as pltpu
from jax.experimental.pallas import tpu_sc as plsc

NUM_CLASSES = 100
CHANNELS = 64
H = 8
W = 8
BATCH = 1024
FEAT = CHANNELS * H * W

CLS_PAD = 104
ROW_BLK = 8


NSC = 2
NSUB = 16
LB = 128


def _sc_argmax_body(lgt_hbm, cls_hbm, lg_buf, cls_buf, in_sem, out_sem):
    c = lax.axis_index("c")
    s = lax.axis_index("s")
    wid = s * NSC + c

    @pl.when(wid < BATCH // LB)
    def _work():
        h_in = pltpu.make_async_copy(
            lgt_hbm.at[:, pl.ds(wid * LB, LB)], lg_buf, in_sem)
        h_in.start()
        h_in.wait()
        for l in range(LB // 16):
            best_v = lg_buf[0, pl.ds(l * 16, 16)]
            best_i = jnp.zeros((16,), jnp.int32)

            def body(cc, carry):
                bv, bi = carry
                v = lg_buf[cc, pl.ds(l * 16, 16)]
                upd = v > bv
                return (jnp.where(upd, v, bv),
                        jnp.where(upd, cc, bi))

            best_v, best_i = lax.fori_loop(
                1, NUM_CLASSES, body, (best_v, best_i), unroll=4)
            cls_buf[0, pl.ds(l * 16, 16)] = best_i
        h_out = pltpu.make_async_copy(cls_buf, cls_hbm.at[wid], out_sem)
        h_out.start()
        h_out.wait()


_sc_argmax = functools.partial(
    pl.kernel,
    out_type=jax.ShapeDtypeStruct((BATCH // LB, 1, LB), jnp.int32),
    mesh=plsc.VectorSubcoreMesh(core_axis_name="c", subcore_axis_name="s"),
    scratch_types=[
        pltpu.VMEM((NUM_CLASSES, LB), jnp.float32),
        pltpu.VMEM((1, LB), jnp.int32),
        pltpu.SemaphoreType.DMA,
        pltpu.SemaphoreType.DMA,
    ],
)(_sc_argmax_body)


def _tc_proto_body(xt_ref, cls_ref, protot_ref, proto_ref):
    ciota = lax.broadcasted_iota(jnp.int32, (CLS_PAD, 1), 0)
    onehot_t = jnp.concatenate(
        [(cls_ref[w] == ciota).astype(jnp.float32) for w in range(BATCH // LB)],
        axis=1)
    xt = xt_ref[...]
    sums_t = lax.dot_general(
        xt, onehot_t,
        dimension_numbers=(((1,), (1,)), ((), ())),
        preferred_element_type=jnp.float32)
    sums = lax.dot_general(
        onehot_t, xt,
        dimension_numbers=(((1,), (1,)), ((), ())),
        preferred_element_type=jnp.float32)
    counts = jnp.sum(onehot_t, axis=1)
    denom = jnp.where(counts > 0, counts, 1.0)
    protot_ref[...] = (sums_t / denom[None, :])[:, :NUM_CLASSES]
    proto_ref[...] = (sums / denom[:, None])[:NUM_CLASSES, :]


def _tc_diff_body(pt_ref, pi_ref, out_ref):
    pt = pt_ref[...]
    pi = pi_ref[...]
    out_ref[...] = pt[None, :, :] - pi[:, :, None]


def kernel(x, class_logits):
    xt = jnp.transpose(x, (1, 2, 3, 0)).reshape(FEAT, BATCH)
    lgt = jnp.transpose(class_logits, (1, 0))
    cls = _sc_argmax(lgt)
    protot, proto = pl.pallas_call(
        _tc_proto_body,
        in_specs=[
            pl.BlockSpec((FEAT, BATCH), lambda: (0, 0)),
            pl.BlockSpec((BATCH // LB, 1, LB), lambda: (0, 0, 0)),
        ],
        out_specs=[
            pl.BlockSpec((FEAT, NUM_CLASSES), lambda: (0, 0)),
            pl.BlockSpec((NUM_CLASSES, FEAT), lambda: (0, 0)),
        ],
        out_shape=[
            jax.ShapeDtypeStruct((FEAT, NUM_CLASSES), jnp.float32),
            jax.ShapeDtypeStruct((NUM_CLASSES, FEAT), jnp.float32),
        ],
    )(xt, cls)

    n_steps = pl.cdiv(NUM_CLASSES, ROW_BLK)
    inter = pl.pallas_call(
        _tc_diff_body,
        grid=(n_steps,),
        in_specs=[
            pl.BlockSpec((FEAT, NUM_CLASSES), lambda b: (0, 0)),
            pl.BlockSpec((ROW_BLK, FEAT), lambda b: (b, 0)),
        ],
        out_specs=pl.BlockSpec((ROW_BLK, FEAT, NUM_CLASSES), lambda b: (b, 0, 0)),
        out_shape=jax.ShapeDtypeStruct(
            (NUM_CLASSES, FEAT, NUM_CLASSES), jnp.float32),
    )(protot, proto)

    prototypes = jnp.transpose(
        protot.reshape(CHANNELS, H, W, NUM_CLASSES), (3, 0, 1, 2))
    inter_class_matrix = jnp.transpose(
        inter.reshape(NUM_CLASSES, CHANNELS, H, W, NUM_CLASSES),
        (0, 4, 1, 2, 3))
    return (prototypes, inter_class_matrix)


# speedup vs baseline: 1.0348x; 1.0348x over previous
import functools

import jax
import jax.numpy as jnp
from jax import lax
from jax.experimental import pallas as pl
from jax.experimental.pallas import tpu as pltpu
from jax.experimental.pallas import tpu_sc as plsc

NUM_CLASSES = 100
CHANNELS = 64
H = 8
W = 8
BATCH = 1024
FEAT = CHANNELS * H * W

CLS_PAD = 104
ROW_BLK = 8


NSC = 2
NSUB = 16
LB = 128


def _sc_argmax_body(lgt_hbm, cls_hbm, lg_buf, cls_buf, in_sem, out_sem):
    c = lax.axis_index("c")
    s = lax.axis_index("s")
    wid = s * NSC + c

    @pl.when(wid < BATCH // LB)
    def _work():
        h_in = pltpu.make_async_copy(
            lgt_hbm.at[:, pl.ds(wid * LB, LB)], lg_buf, in_sem)
        h_in.start()
        h_in.wait()
        for l in range(LB // 16):
            best_v = lg_buf[0, pl.ds(l * 16, 16)]
            best_i = jnp.zeros((16,), jnp.int32)

            def body(cc, carry):
                bv, bi = carry
                v = lg_buf[cc, pl.ds(l * 16, 16)]
                upd = v > bv
                return (jnp.where(upd, v, bv),
                        jnp.where(upd, cc, bi))

            best_v, best_i = lax.fori_loop(
                1, NUM_CLASSES, body, (best_v, best_i), unroll=4)
            cls_buf[0, pl.ds(l * 16, 16)] = best_i
        h_out = pltpu.make_async_copy(cls_buf, cls_hbm.at[wid], out_sem)
        h_out.start()
        h_out.wait()


_sc_argmax = functools.partial(
    pl.kernel,
    out_type=jax.ShapeDtypeStruct((BATCH // LB, 1, LB), jnp.int32),
    mesh=plsc.VectorSubcoreMesh(core_axis_name="c", subcore_axis_name="s"),
    scratch_types=[
        pltpu.VMEM((NUM_CLASSES, LB), jnp.float32),
        pltpu.VMEM((1, LB), jnp.int32),
        pltpu.SemaphoreType.DMA,
        pltpu.SemaphoreType.DMA,
    ],
)(_sc_argmax_body)


def _tc_fused_body(xt_ref, cls_ref, protot_out_ref, inter_ref,
                   protot_scr, proto_scr):
    b = pl.program_id(0)

    @pl.when(b == 0)
    def _init():
        ciota = lax.broadcasted_iota(jnp.int32, (CLS_PAD, 1), 0)
        onehot_t = jnp.concatenate(
            [(cls_ref[w] == ciota).astype(jnp.float32)
             for w in range(BATCH // LB)],
            axis=1)
        xt = xt_ref[...]
        sums_t = lax.dot_general(
            xt, onehot_t,
            dimension_numbers=(((1,), (1,)), ((), ())),
            preferred_element_type=jnp.float32)
        sums = lax.dot_general(
            onehot_t, xt,
            dimension_numbers=(((1,), (1,)), ((), ())),
            preferred_element_type=jnp.float32)
        counts = jnp.sum(onehot_t, axis=1)
        denom = jnp.where(counts > 0, counts, 1.0)
        protot = sums_t / denom[None, :]
        protot_scr[...] = protot
        proto_scr[...] = sums / denom[:, None]
        protot_out_ref[...] = protot[:, :NUM_CLASSES]

    pt = protot_scr[...][:, :NUM_CLASSES]
    pi = proto_scr[pl.ds(b * ROW_BLK, ROW_BLK), :]
    inter_ref[...] = pt[None, :, :] - pi[:, :, None]


def kernel(x, class_logits):
    xt = jnp.transpose(x, (1, 2, 3, 0)).reshape(FEAT, BATCH)
    lgt = jnp.transpose(class_logits, (1, 0))
    cls = _sc_argmax(lgt)
    n_steps = pl.cdiv(NUM_CLASSES, ROW_BLK)
    protot, inter = pl.pallas_call(
        _tc_fused_body,
        grid=(n_steps,),
        in_specs=[
            pl.BlockSpec((FEAT, BATCH), lambda b: (0, 0)),
            pl.BlockSpec((BATCH // LB, 1, LB), lambda b: (0, 0, 0)),
        ],
        out_specs=[
            pl.BlockSpec((FEAT, NUM_CLASSES), lambda b: (0, 0)),
            pl.BlockSpec((ROW_BLK, FEAT, NUM_CLASSES), lambda b: (b, 0, 0)),
        ],
        out_shape=[
            jax.ShapeDtypeStruct((FEAT, NUM_CLASSES), jnp.float32),
            jax.ShapeDtypeStruct((NUM_CLASSES, FEAT, NUM_CLASSES), jnp.float32),
        ],
        scratch_shapes=[
            pltpu.VMEM((FEAT, CLS_PAD), jnp.float32),
            pltpu.VMEM((CLS_PAD, FEAT), jnp.float32),
        ],
    )(xt, cls)

    prototypes = jnp.transpose(
        protot.reshape(CHANNELS, H, W, NUM_CLASSES), (3, 0, 1, 2))
    inter_class_matrix = jnp.transpose(
        inter.reshape(NUM_CLASSES, CHANNELS, H, W, NUM_CLASSES),
        (0, 4, 1, 2, 3))
    return (prototypes, inter_class_matrix)


# speedup vs baseline: 1.0497x; 1.0143x over previous
import functools

import jax
import jax.numpy as jnp
from jax import lax
from jax.experimental import pallas as pl
from jax.experimental.pallas import tpu as pltpu
from jax.experimental.pallas import tpu_sc as plsc

NUM_CLASSES = 100
CHANNELS = 64
H = 8
W = 8
BATCH = 1024
FEAT = CHANNELS * H * W

CLS_PAD = 104
ROW_BLK = 8


NSC = 2
NSUB = 16
LB = 128


def _sc_argmax_body(lgt_hbm, cls_hbm, lg_buf, cls_buf, in_sem, out_sem):
    wid = lax.axis_index("s")

    @pl.when(wid < BATCH // LB)
    def _work():
        h_in = pltpu.make_async_copy(
            lgt_hbm.at[:, pl.ds(wid * LB, LB)], lg_buf, in_sem)
        h_in.start()
        h_in.wait()
        for l in range(LB // 16):
            best_v = lg_buf[0, pl.ds(l * 16, 16)]
            best_i = jnp.zeros((16,), jnp.int32)

            def body(cc, carry):
                bv, bi = carry
                v = lg_buf[cc, pl.ds(l * 16, 16)]
                upd = v > bv
                return (jnp.where(upd, v, bv),
                        jnp.where(upd, cc, bi))

            best_v, best_i = lax.fori_loop(
                1, NUM_CLASSES, body, (best_v, best_i), unroll=4)
            cls_buf[0, pl.ds(l * 16, 16)] = best_i
        h_out = pltpu.make_async_copy(cls_buf, cls_hbm.at[wid], out_sem)
        h_out.start()
        h_out.wait()


_sc_argmax = functools.partial(
    pl.kernel,
    out_type=jax.ShapeDtypeStruct((BATCH // LB, 1, LB), jnp.int32),
    mesh=plsc.VectorSubcoreMesh(
        core_axis_name="c", subcore_axis_name="s", num_cores=1),
    scratch_types=[
        pltpu.VMEM((NUM_CLASSES, LB), jnp.float32),
        pltpu.VMEM((1, LB), jnp.int32),
        pltpu.SemaphoreType.DMA,
        pltpu.SemaphoreType.DMA,
    ],
)(_sc_argmax_body)


def _tc_fused_body(xt_ref, cls_ref, protot_out_ref, inter_ref,
                   protot_scr, proto_scr):
    b = pl.program_id(0)

    @pl.when(b == 0)
    def _init():
        ciota = lax.broadcasted_iota(jnp.int32, (CLS_PAD, 1), 0)
        onehot_t = jnp.concatenate(
            [(cls_ref[w] == ciota).astype(jnp.float32)
             for w in range(BATCH // LB)],
            axis=1)
        xt = xt_ref[...]
        sums_t = lax.dot_general(
            xt, onehot_t,
            dimension_numbers=(((1,), (1,)), ((), ())),
            preferred_element_type=jnp.float32)
        sums = lax.dot_general(
            onehot_t, xt,
            dimension_numbers=(((1,), (1,)), ((), ())),
            preferred_element_type=jnp.float32)
        counts = jnp.sum(onehot_t, axis=1)
        denom = jnp.where(counts > 0, counts, 1.0)
        protot = sums_t / denom[None, :]
        protot_scr[...] = protot
        proto_scr[...] = sums / denom[:, None]
        protot_out_ref[...] = protot[:, :NUM_CLASSES]

    pt = protot_scr[...][:, :NUM_CLASSES]
    pi = proto_scr[pl.ds(b * ROW_BLK, ROW_BLK), :]
    inter_ref[...] = pt[None, :, :] - pi[:, :, None]


def kernel(x, class_logits):
    xt = jnp.transpose(x, (1, 2, 3, 0)).reshape(FEAT, BATCH)
    lgt = jnp.transpose(class_logits, (1, 0))
    cls = _sc_argmax(lgt)
    n_steps = pl.cdiv(NUM_CLASSES, ROW_BLK)
    protot, inter = pl.pallas_call(
        _tc_fused_body,
        grid=(n_steps,),
        in_specs=[
            pl.BlockSpec((FEAT, BATCH), lambda b: (0, 0)),
            pl.BlockSpec((BATCH // LB, 1, LB), lambda b: (0, 0, 0)),
        ],
        out_specs=[
            pl.BlockSpec((FEAT, NUM_CLASSES), lambda b: (0, 0)),
            pl.BlockSpec((ROW_BLK, FEAT, NUM_CLASSES), lambda b: (b, 0, 0)),
        ],
        out_shape=[
            jax.ShapeDtypeStruct((FEAT, NUM_CLASSES), jnp.float32),
            jax.ShapeDtypeStruct((NUM_CLASSES, FEAT, NUM_CLASSES), jnp.float32),
        ],
        scratch_shapes=[
            pltpu.VMEM((FEAT, CLS_PAD), jnp.float32),
            pltpu.VMEM((CLS_PAD, FEAT), jnp.float32),
        ],
    )(xt, cls)

    prototypes = jnp.transpose(
        protot.reshape(CHANNELS, H, W, NUM_CLASSES), (3, 0, 1, 2))
    inter_class_matrix = jnp.transpose(
        inter.reshape(NUM_CLASSES, CHANNELS, H, W, NUM_CLASSES),
        (0, 4, 1, 2, 3))
    return (prototypes, inter_class_matrix)
